# baseline (speedup 1.0000x reference)
import functools

import jax
import jax.numpy as jnp
from jax import lax
from jax.experimental import pallas as pl
from jax.experimental.pallas import tpu as pltpu
from jax.experimental.pallas import tpu_sc as plsc

NC = 2
NS = 16
NW = NC * NS
EB = 128


RB = 2
NP = 2 * RB


def _sc_degree(dstp, zed, ones, n_pad, e_work):
    epw = e_work // NW
    nb = epw // EB
    rpt = n_pad // NS
    mesh = plsc.VectorSubcoreMesh(core_axis_name="c", subcore_axis_name="s", num_cores=NC, num_subcores=NS)

    @functools.partial(
        pl.kernel,
        out_type=jax.ShapeDtypeStruct((NC, n_pad, 16), jnp.float32),
        mesh=mesh,
        scratch_types=[
            pltpu.VMEM((EB,), jnp.int32),
            pltpu.VMEM((EB, 16), jnp.float32),
            pltpu.VMEM_SHARED((n_pad, 16), jnp.float32),
        ],
    )
    def k(dst_hbm, zed_hbm, ones_hbm, out_hbm, idx_v, ones_v, deg_sh):
        c = lax.axis_index("c")
        s = lax.axis_index("s")
        wid = s * NC + c
        pltpu.sync_copy(ones_hbm, ones_v)
        pltpu.sync_copy(zed_hbm.at[pl.ds(s * rpt, rpt)],
                        deg_sh.at[pl.ds(s * rpt, rpt)])
        plsc.subcore_barrier()

        def step(b, carry):
            base = wid * epw + b * EB
            pltpu.sync_copy(dst_hbm.at[pl.ds(base, EB)], idx_v)
            pltpu.sync_copy(ones_v, deg_sh.at[idx_v], add=True)
            return carry

        lax.fori_loop(0, nb, step, 0)
        plsc.subcore_barrier()
        pltpu.sync_copy(deg_sh.at[pl.ds(s * rpt, rpt)],
                        out_hbm.at[c, pl.ds(s * rpt, rpt)])

    return k(dstp, zed, ones)


def _sc_spmm(g, srcp, dstp, zacc, n_pad, e_work):
    d = g.shape[1]
    epw = e_work // NW
    nb = epw // EB
    rpt = n_pad // NS
    mesh = plsc.VectorSubcoreMesh(core_axis_name="c", subcore_axis_name="s", num_cores=NC, num_subcores=NS)

    @functools.partial(
        pl.kernel,
        out_type=jax.ShapeDtypeStruct((NC, n_pad, d), jnp.float32),
        mesh=mesh,
        scratch_types=[
            [pltpu.VMEM((EB,), jnp.int32)] * NP,
            [pltpu.VMEM((EB,), jnp.int32)] * NP,
            [pltpu.VMEM((EB, d), jnp.float32)] * RB,
            [pltpu.SemaphoreType.DMA] * NP,
            [pltpu.SemaphoreType.DMA] * RB,
            [pltpu.SemaphoreType.DMA] * RB,
            pltpu.VMEM_SHARED((n_pad, d), jnp.float32),
        ],
    )
    def k(g_hbm, src_hbm, dst_hbm, zacc_hbm, out_hbm,
          idx_s, idx_d, bufs, sem_i, sem_g, sem_s, acc_sh):
        c = lax.axis_index("c")
        s = lax.axis_index("s")
        wid = s * NC + c
        base = wid * epw

        def istart(b, p):
            pltpu.async_copy(src_hbm.at[pl.ds(base + b * EB, EB)],
                             idx_s[p], sem_i[p])
            pltpu.async_copy(dst_hbm.at[pl.ds(base + b * EB, EB)],
                             idx_d[p], sem_i[p])

        def iwait(p):
            pltpu.make_async_copy(src_hbm.at[pl.ds(base, EB)],
                                  idx_s[p], sem_i[p]).wait()
            pltpu.make_async_copy(dst_hbm.at[pl.ds(base, EB)],
                                  idx_d[p], sem_i[p]).wait()

        def gstart(p, r):
            pltpu.async_copy(g_hbm.at[idx_s[p]], bufs[r], sem_g[r])

        def gwait(p, r):
            pltpu.make_async_copy(g_hbm.at[idx_s[p]], bufs[r],
                                  sem_g[r]).wait()

        def sstart(p, r):
            pltpu.async_copy(bufs[r], acc_sh.at[idx_d[p]], sem_s[r], add=True)

        def swait(p, r):
            pltpu.make_async_copy(bufs[r], acc_sh.at[idx_d[p]],
                                  sem_s[r]).wait()

        pltpu.sync_copy(zacc_hbm.at[pl.ds(s * rpt, rpt)],
                        acc_sh.at[pl.ds(s * rpt, rpt)])
        plsc.subcore_barrier()

        for r in range(RB):
            istart(r, r)
            pltpu.sync_copy(dst_hbm.at[pl.ds(base, EB)], idx_d[RB + r])
            pltpu.sync_copy(zacc_hbm.at[pl.ds(0, EB)], bufs[r])
            sstart(RB + r, r)

        def cycle(bbase, pc, pn):
            for r in range(RB):
                swait(pn + r, r)
                iwait(pc + r)
                gstart(pc + r, r)
            for r in range(RB):
                gwait(pc + r, r)
                sstart(pc + r, r)
            for r in range(RB):
                istart(bbase + RB + r, pn + r)

        def step(i, carry):
            bb = i * NP
            cycle(bb, 0, RB)
            cycle(bb + RB, RB, 0)
            return carry

        lax.fori_loop(0, nb // NP, step, 0)

        for r in range(RB):
            swait(RB + r, r)
            iwait(r)
        plsc.subcore_barrier()
        pltpu.sync_copy(acc_sh.at[pl.ds(s * rpt, rpt)],
                        out_hbm.at[c, pl.ds(s * rpt, rpt)])

    return k(g, srcp, dstp, zacc)


def _dis_block(dg_ref):
    return lax.rsqrt(dg_ref[0, :, 0:1] + dg_ref[1, :, 0:1] + 1.0)


def _tc_scale_matmul(x, w, deg16, bm):
    n, d = x.shape

    def body(x_ref, w_ref, dg_ref, out_ref):
        dis = _dis_block(dg_ref)
        out_ref[...] = dis * jnp.dot(x_ref[...], w_ref[...],
                                     preferred_element_type=jnp.float32)

    return pl.pallas_call(
        body,
        grid=(n // bm,),
        in_specs=[
            pl.BlockSpec((bm, d), lambda i: (i, 0)),
            pl.BlockSpec((d, d), lambda i: (0, 0)),
            pl.BlockSpec((2, bm, 16), lambda i: (0, i, 0)),
        ],
        out_specs=pl.BlockSpec((bm, d), lambda i: (i, 0)),
        out_shape=jax.ShapeDtypeStruct((n, d), jnp.float32),
    )(x, w, deg16)


def _tc_mid(acc, g1, deg16, b1, w2, bm):
    n, d = g1.shape

    def body(acc_ref, g1_ref, dg_ref, b1_ref, w2_ref, out_ref):
        dis = _dis_block(dg_ref)
        o1 = dis * (acc_ref[0] + acc_ref[1] + g1_ref[...]) + b1_ref[...]
        o1 = jnp.maximum(o1, 0.0)
        out_ref[...] = dis * jnp.dot(o1, w2_ref[...],
                                     preferred_element_type=jnp.float32)

    return pl.pallas_call(
        body,
        grid=(n // bm,),
        in_specs=[
            pl.BlockSpec((2, bm, d), lambda i: (0, i, 0)),
            pl.BlockSpec((bm, d), lambda i: (i, 0)),
            pl.BlockSpec((2, bm, 16), lambda i: (0, i, 0)),
            pl.BlockSpec((1, d), lambda i: (0, 0)),
            pl.BlockSpec((d, d), lambda i: (0, 0)),
        ],
        out_specs=pl.BlockSpec((bm, d), lambda i: (i, 0)),
        out_shape=jax.ShapeDtypeStruct((n, d), jnp.float32),
    )(acc, g1, deg16, b1, w2)


def _tc_final(acc, g2, deg16, b2, bm):
    n, d = g2.shape

    def body(acc_ref, g2_ref, dg_ref, b2_ref, out_ref):
        dis = _dis_block(dg_ref)
        out_ref[...] = dis * (acc_ref[0] + acc_ref[1] + g2_ref[...]) + b2_ref[...]

    return pl.pallas_call(
        body,
        grid=(n // bm,),
        in_specs=[
            pl.BlockSpec((2, bm, d), lambda i: (0, i, 0)),
            pl.BlockSpec((bm, d), lambda i: (i, 0)),
            pl.BlockSpec((2, bm, 16), lambda i: (0, i, 0)),
            pl.BlockSpec((1, d), lambda i: (0, 0)),
        ],
        out_specs=pl.BlockSpec((bm, d), lambda i: (i, 0)),
        out_shape=jax.ShapeDtypeStruct((n, d), jnp.float32),
    )(acc, g2, deg16, b2)


def kernel(x, edge_index, W1, b1, W2, b2):
    n, d = x.shape
    e = edge_index.shape[1]
    src = edge_index[0].astype(jnp.int32)
    dst = edge_index[1].astype(jnp.int32)

    chunk = NW * EB * NP
    e_work = -(-e // chunk) * chunk
    pad = e_work - e
    srcp = jnp.concatenate([src, jnp.zeros((pad,), jnp.int32),
                            jnp.zeros((RB * EB,), jnp.int32)])
    dstp = jnp.concatenate([dst, jnp.full((pad,), n, jnp.int32),
                            jnp.zeros((RB * EB,), jnp.int32)])

    n_pad = -(-(n + 1) // (NS * 8)) * (NS * 8)
    zed = jnp.zeros((n_pad, 16), jnp.float32)
    ones = jnp.ones((EB, 16), jnp.float32)
    zacc = jnp.zeros((n_pad, d), jnp.float32)

    deg16 = _sc_degree(dstp, zed, ones, n_pad, e_work)[:, :n]

    bm = 1000 if n % 1000 == 0 else 8
    b1r = b1.reshape(1, d)
    b2r = b2.reshape(1, d)

    g1 = _tc_scale_matmul(x, W1, deg16, bm)
    acc1 = _sc_spmm(g1, srcp, dstp, zacc, n_pad, e_work)[:, :n]
    g2 = _tc_mid(acc1, g1, deg16, b1r, W2, bm)
    acc2 = _sc_spmm(g2, srcp, dstp, zacc, n_pad, e_work)[:, :n]
    return _tc_final(acc2, g2, deg16, b2r, bm)
